# baseline (speedup 1.0000x reference)
import functools

import jax
import jax.numpy as jnp
from jax import lax
from jax.experimental import pallas as pl
from jax.experimental.pallas import tpu as pltpu
from jax.experimental.pallas import tpu_sc as plsc

B = 2
NV = 6890
RES = 128
VOL = RES * RES * RES
SIG2 = 0.05 * 0.05
NS = 16
VPS = 432
NVPAD = NS * VPS
NVREG = VPS // 16
NCON = VPS * 27
NCHUNK = (NCON + 127) // 128
CPAD = NCHUNK * 128
GRP = 4
NGRP = NCHUNK // GRP
GROWS = GRP * 128
RING = 6
NPASSES = 8
PROWS = VOL // 2 // NPASSES
SROWS = PROWS // NS
EPR = 256
EPV = EPR * 2
NEP = SROWS // EPR
ZR = 2048

_OFFS = [(a, b, c) for a in (-1, 0, 1) for b in (-1, 0, 1) for c in (-1, 0, 1)]


def _floor_i32(x):
    t = x.astype(jnp.int32)
    return t - jnp.where(x < t.astype(jnp.float32), 1, 0).astype(jnp.int32)


def _sc_body(vx, vy, vz, c0, c1, c2, zeros_in, out,
             px, py, pz, q0, q1, q2, rows_all, stage, vbuf, lidx, acc_in,
             out_buf, vals_hbm, psem, ssem, zsem, esem, osem, acc):
    core = lax.axis_index("c")
    sub = lax.axis_index("s")
    wid = core * NS + sub
    vbase = core * NVPAD + sub * VPS
    iota = lax.iota(jnp.int32, 16)
    i2 = lax.shift_right_logical(iota, 1)
    p4 = (iota & 1) * 4

    pltpu.sync_copy(vx.at[pl.ds(vbase, VPS)], px)
    pltpu.sync_copy(vy.at[pl.ds(vbase, VPS)], py)
    pltpu.sync_copy(vz.at[pl.ds(vbase, VPS)], pz)
    pltpu.sync_copy(c0.at[pl.ds(vbase, VPS)], q0)
    pltpu.sync_copy(c1.at[pl.ds(vbase, VPS)], q1)
    pltpu.sync_copy(c2.at[pl.ds(vbase, VPS)], q2)

    for m in range((CPAD - NCON) // 16):
        rows_all[pl.ds(NCON + m * 16, 16)] = jnp.full((16,), 1 << 29,
                                                      jnp.int32)
    pltpu.sync_copy(zeros_in.at[pl.ds(0, CPAD - NCON)],
                    vals_hbm.at[wid, pl.ds(NCON, CPAD - NCON)])

    def gen(i, _):
        r16 = i * 16
        wx = px[pl.ds(r16, 16)]
        wy = py[pl.ds(r16, 16)]
        wz = pz[pl.ds(r16, 16)]
        a0 = q0[pl.ds(r16, 16)]
        a1 = q1[pl.ds(r16, 16)]
        a2 = q2[pl.ds(r16, 16)]
        bx = _floor_i32((wx * 0.5 + 0.5) * RES)
        by = _floor_i32((wy * 0.5 + 0.5) * RES)
        bz = _floor_i32((wz * 0.5 + 0.5) * RES)
        zero = jnp.zeros((16,), jnp.float32)
        for o, (oa, ob, oc) in enumerate(_OFFS):
            nx = jnp.clip(bx + oa, 0, RES - 1)
            ny = jnp.clip(by + ob, 0, RES - 1)
            nz = jnp.clip(bz + oc, 0, RES - 1)
            dx = (nx.astype(jnp.float32) + 0.5) * (2.0 / RES) - 1.0 - wx
            dy = (ny.astype(jnp.float32) + 0.5) * (2.0 / RES) - 1.0 - wy
            dz = (nz.astype(jnp.float32) + 0.5) * (2.0 / RES) - 1.0 - wz
            w = jnp.exp((dx * dx + dy * dy + dz * dz) * (-1.0 / SIG2))
            g = (nx * RES + ny) * RES + nz
            rows_all[pl.ds(i * VPS + o * 16, 16)] = (
                lax.shift_right_logical(g, 1))
            rvec = o * 16 + iota
            half = (g & 1) * 4
            anti = 4 - half
            for ch, val in enumerate((w * a0, w * a1, w * a2, w)):
                plsc.store_scatter(stage, [rvec, half + ch], val)
                plsc.store_scatter(stage, [rvec, anti + ch], zero)
        pltpu.sync_copy(stage, vals_hbm.at[wid, pl.ds(i * VPS, VPS)])
        return 0

    lax.fori_loop(0, NVREG, gen, 0)

    def one_pass(p, _):
        row_lo = p * PROWS
        for k in range(SROWS // ZR):
            pltpu.async_copy(zeros_in,
                             acc.at[pl.ds(sub * SROWS + k * ZR, ZR)], zsem)
        for k in range(SROWS // ZR):
            pltpu.make_async_copy(
                zeros_in, acc.at[pl.ds(sub * SROWS + k * ZR, ZR)],
                zsem).wait()

        def remap(j, _):
            for k in range(8):
                r = rows_all[pl.ds(j * 128 + k * 16, 16)]
                rel = r - row_lo
                match = (rel >= 0) & (rel < PROWS)
                trash = PROWS + k * 16 + iota
                lidx[j, pl.ds(k * 16, 16)] = jnp.where(match, rel, trash)
            return 0

        lax.fori_loop(0, NCHUNK, remap, 0)
        plsc.subcore_barrier()

        for r in range(RING - 1):
            pltpu.async_copy(vals_hbm.at[wid, pl.ds(r * GROWS, GROWS)],
                             vbuf.at[r], psem)

        def scat_grp(g, _):
            bi = lax.rem(g, RING)
            pltpu.make_async_copy(vals_hbm.at[wid, pl.ds(g * GROWS, GROWS)],
                                  vbuf.at[bi], psem).wait()
            for cc in range(GRP):
                pltpu.async_copy(vbuf.at[bi, pl.ds(cc * 128, 128)],
                                 acc.at[lidx.at[g * GRP + cc]], ssem,
                                 add=True)

            @pl.when(g >= 1)
            def _():
                pg = g - 1
                pbi = lax.rem(pg, RING)
                for cc in range(GRP):
                    pltpu.make_async_copy(
                        vbuf.at[pbi, pl.ds(cc * 128, 128)],
                        acc.at[lidx.at[pg * GRP + cc]], ssem).wait()

            @pl.when(g + RING - 1 < NGRP)
            def _():
                pltpu.async_copy(
                    vals_hbm.at[wid, pl.ds((g + RING - 1) * GROWS, GROWS)],
                    vbuf.at[lax.rem(g + RING - 1, RING)], psem)
            return 0

        lax.fori_loop(0, NGRP, scat_grp, 0)
        for cc in range(GRP):
            pltpu.make_async_copy(
                vbuf.at[(NGRP - 1) % RING, pl.ds(cc * 128, 128)],
                acc.at[lidx.at[(NGRP - 1) * GRP + cc]], ssem).wait()
        plsc.subcore_barrier()

        sbase = sub * SROWS

        def _ep_in(k, bi):
            pltpu.async_copy(acc.at[pl.ds(sbase + k * EPR, EPR)],
                             acc_in.at[bi], esem)

        def _ep_out(k, bi, start):
            vox = (row_lo + sbase + k * EPR) * 2
            for ch in range(3):
                d = pltpu.make_async_copy(
                    out_buf.at[bi, pl.ds(ch * EPV, EPV)],
                    out.at[pl.ds((core * 3 + ch) * VOL + vox, EPV)], osem)
                if start:
                    d.start()
                else:
                    d.wait()

        _ep_in(0, 0)

        def epi(k, _):
            bi = lax.rem(k, 2)
            pltpu.make_async_copy(acc.at[pl.ds(sbase + k * EPR, EPR)],
                                  acc_in.at[bi], esem).wait()

            @pl.when(k + 1 < NEP)
            def _():
                _ep_in(k + 1, lax.rem(k + 1, 2))

            @pl.when(k >= 2)
            def _():
                _ep_out(k - 2, bi, False)

            src = acc_in.at[bi]
            for m in range(EPV // 16):
                rr = m * 8 + i2
                ws = plsc.load_gather(src, [rr, p4 + 3])
                inv = 1.0 / (ws + 0.001)
                for ch in range(3):
                    s = plsc.load_gather(src, [rr, p4 + ch])
                    out_buf[bi, pl.ds(ch * EPV + m * 16, 16)] = s * inv
            _ep_out(k, bi, True)
            return 0

        lax.fori_loop(0, NEP, epi, 0)
        _ep_out(NEP - 2, 0, False)
        _ep_out(NEP - 1, 1, False)
        plsc.subcore_barrier()
        return 0

    lax.fori_loop(0, NPASSES, one_pass, 0)


_sc_vox = functools.partial(
    pl.kernel,
    out_type=jax.ShapeDtypeStruct((B * 3 * VOL,), jnp.float32),
    mesh=plsc.VectorSubcoreMesh(core_axis_name="c", subcore_axis_name="s"),
    compiler_params=pltpu.CompilerParams(needs_layout_passes=False,
                                         use_tc_tiling_on_sc=False),
    scratch_types=[
        pltpu.VMEM((VPS,), jnp.float32),
        pltpu.VMEM((VPS,), jnp.float32),
        pltpu.VMEM((VPS,), jnp.float32),
        pltpu.VMEM((VPS,), jnp.float32),
        pltpu.VMEM((VPS,), jnp.float32),
        pltpu.VMEM((VPS,), jnp.float32),
        pltpu.VMEM((CPAD,), jnp.int32),
        pltpu.VMEM((VPS, 8), jnp.float32),
        pltpu.VMEM((RING, GROWS, 8), jnp.float32),
        pltpu.VMEM((NCHUNK, 128), jnp.int32),
        pltpu.VMEM((2, EPR, 8), jnp.float32),
        pltpu.VMEM((2, 3 * EPV), jnp.float32),
        pltpu.HBM((B * NS, CPAD, 8), jnp.float32),
        pltpu.SemaphoreType.DMA,
        pltpu.SemaphoreType.DMA,
        pltpu.SemaphoreType.DMA,
        pltpu.SemaphoreType.DMA,
        pltpu.SemaphoreType.DMA,
        pltpu.VMEM_SHARED((PROWS + 128, 8), jnp.float32),
    ],
)(_sc_body)


def kernel(smpl_vertices, smpl_vertex_code, smpl_face_code,
           smpl_face_indices, smpl_tetrahedron_indices):
    del smpl_face_code, smpl_face_indices, smpl_tetrahedron_indices
    pad = NVPAD - NV
    v = jnp.pad(smpl_vertices, ((0, 0), (0, pad), (0, 0)),
                constant_values=1e6)
    cde = jnp.pad(smpl_vertex_code, ((0, 0), (0, pad), (0, 0)))
    zeros_in = jnp.zeros((ZR, 8), jnp.float32)
    out = _sc_vox(v[:, :, 0].reshape(-1), v[:, :, 1].reshape(-1),
                  v[:, :, 2].reshape(-1), cde[:, :, 0].reshape(-1),
                  cde[:, :, 1].reshape(-1), cde[:, :, 2].reshape(-1),
                  zeros_in)
    return out.reshape(B, 3, RES, RES, RES)


# speedup vs baseline: 1.0025x; 1.0025x over previous
import functools

import jax
import jax.numpy as jnp
from jax import lax
from jax.experimental import pallas as pl
from jax.experimental.pallas import tpu as pltpu
from jax.experimental.pallas import tpu_sc as plsc

B = 2
NV = 6890
RES = 128
VOL = RES * RES * RES
SIG2 = 0.05 * 0.05
NS = 16
VPS = 432
NVPAD = NS * VPS
NVREG = VPS // 16
NCON = VPS * 27
NCHUNK = (NCON + 127) // 128
CPAD = NCHUNK * 128
GRP = 4
NGRP = NCHUNK // GRP
GROWS = GRP * 128
RING = 6
NPASSES = 8
PROWS = VOL // 2 // NPASSES
SROWS = PROWS // NS
EPR = 256
EPV = EPR * 2
NEP = SROWS // EPR
ZR = 2048

_OFFS = [(a, b, c) for a in (-1, 0, 1) for b in (-1, 0, 1) for c in (-1, 0, 1)]


def _floor_i32(x):
    t = x.astype(jnp.int32)
    return t - jnp.where(x < t.astype(jnp.float32), 1, 0).astype(jnp.int32)


def _sc_body(vx, vy, vz, c0, c1, c2, zeros_in, out,
             px, py, pz, q0, q1, q2, rows_all, stage, vbuf, lidx, acc_in,
             out_buf, vals_hbm, psem, ssem, zsem, esem, osem, acc):
    core = lax.axis_index("c")
    sub = lax.axis_index("s")
    wid = core * NS + sub
    vbase = core * NVPAD + sub * VPS
    iota = lax.iota(jnp.int32, 16)
    i2 = lax.shift_right_logical(iota, 1)
    p4 = (iota & 1) * 4

    pltpu.sync_copy(vx.at[pl.ds(vbase, VPS)], px)
    pltpu.sync_copy(vy.at[pl.ds(vbase, VPS)], py)
    pltpu.sync_copy(vz.at[pl.ds(vbase, VPS)], pz)
    pltpu.sync_copy(c0.at[pl.ds(vbase, VPS)], q0)
    pltpu.sync_copy(c1.at[pl.ds(vbase, VPS)], q1)
    pltpu.sync_copy(c2.at[pl.ds(vbase, VPS)], q2)

    for m in range((CPAD - NCON) // 16):
        rows_all[pl.ds(NCON + m * 16, 16)] = jnp.full((16,), 1 << 29,
                                                      jnp.int32)
    pltpu.sync_copy(zeros_in.at[pl.ds(0, CPAD - NCON)],
                    vals_hbm.at[wid, pl.ds(NCON, CPAD - NCON)])

    def gen(i, _):
        r16 = i * 16
        wx = px[pl.ds(r16, 16)]
        wy = py[pl.ds(r16, 16)]
        wz = pz[pl.ds(r16, 16)]
        a0 = q0[pl.ds(r16, 16)]
        a1 = q1[pl.ds(r16, 16)]
        a2 = q2[pl.ds(r16, 16)]
        bx = _floor_i32((wx * 0.5 + 0.5) * RES)
        by = _floor_i32((wy * 0.5 + 0.5) * RES)
        bz = _floor_i32((wz * 0.5 + 0.5) * RES)
        zero = jnp.zeros((16,), jnp.float32)
        for o, (oa, ob, oc) in enumerate(_OFFS):
            nx = jnp.clip(bx + oa, 0, RES - 1)
            ny = jnp.clip(by + ob, 0, RES - 1)
            nz = jnp.clip(bz + oc, 0, RES - 1)
            dx = (nx.astype(jnp.float32) + 0.5) * (2.0 / RES) - 1.0 - wx
            dy = (ny.astype(jnp.float32) + 0.5) * (2.0 / RES) - 1.0 - wy
            dz = (nz.astype(jnp.float32) + 0.5) * (2.0 / RES) - 1.0 - wz
            w = jnp.exp((dx * dx + dy * dy + dz * dz) * (-1.0 / SIG2))
            g = (nx * RES + ny) * RES + nz
            rows_all[pl.ds(i * VPS + o * 16, 16)] = (
                lax.shift_right_logical(g, 1))
            rvec = o * 16 + iota
            half = (g & 1) * 4
            anti = 4 - half
            for ch, val in enumerate((w * a0, w * a1, w * a2, w)):
                plsc.store_scatter(stage, [rvec, half + ch], val)
                plsc.store_scatter(stage, [rvec, anti + ch], zero)
        pltpu.sync_copy(stage, vals_hbm.at[wid, pl.ds(i * VPS, VPS)])
        return 0

    lax.fori_loop(0, NVREG, gen, 0)

    def one_pass(p, _):
        row_lo = p * PROWS
        for k in range(SROWS // ZR):
            pltpu.async_copy(zeros_in,
                             acc.at[pl.ds(sub * SROWS + k * ZR, ZR)], zsem)
        for k in range(SROWS // ZR):
            pltpu.make_async_copy(
                zeros_in, acc.at[pl.ds(sub * SROWS + k * ZR, ZR)],
                zsem).wait()

        def remap(j, _):
            for k in range(8):
                r = rows_all[pl.ds(j * 128 + k * 16, 16)]
                rel = r - row_lo
                match = (rel >= 0) & (rel < PROWS)
                trash = PROWS + k * 16 + iota
                lidx[j, pl.ds(k * 16, 16)] = jnp.where(match, rel, trash)
            return 0

        lax.fori_loop(0, NCHUNK, remap, 0)
        plsc.subcore_barrier()

        for r in range(RING - 1):
            pltpu.async_copy(vals_hbm.at[wid, pl.ds(r * GROWS, GROWS)],
                             vbuf.at[r], psem)

        def scat_grp(g, _):
            bi = lax.rem(g, RING)
            pltpu.make_async_copy(vals_hbm.at[wid, pl.ds(g * GROWS, GROWS)],
                                  vbuf.at[bi], psem).wait()
            for cc in range(GRP):
                pltpu.async_copy(vbuf.at[bi, pl.ds(cc * 128, 128)],
                                 acc.at[lidx.at[g * GRP + cc]], ssem,
                                 add=True)

            @pl.when(g >= 1)
            def _():
                pg = g - 1
                pbi = lax.rem(pg, RING)
                for cc in range(GRP):
                    pltpu.make_async_copy(
                        vbuf.at[pbi, pl.ds(cc * 128, 128)],
                        acc.at[lidx.at[pg * GRP + cc]], ssem).wait()

            @pl.when(g + RING - 1 < NGRP)
            def _():
                pltpu.async_copy(
                    vals_hbm.at[wid, pl.ds((g + RING - 1) * GROWS, GROWS)],
                    vbuf.at[lax.rem(g + RING - 1, RING)], psem)
            return 0

        lax.fori_loop(0, NGRP, scat_grp, 0)
        for cc in range(GRP):
            pltpu.make_async_copy(
                vbuf.at[(NGRP - 1) % RING, pl.ds(cc * 128, 128)],
                acc.at[lidx.at[(NGRP - 1) * GRP + cc]], ssem).wait()
        plsc.subcore_barrier()

        sbase = sub * SROWS

        def _ep_in(k, bi):
            pltpu.async_copy(acc.at[pl.ds(sbase + k * EPR, EPR)],
                             acc_in.at[bi], esem)

        def _ep_out(k, bi, start):
            vox = (row_lo + sbase + k * EPR) * 2
            for ch in range(3):
                d = pltpu.make_async_copy(
                    out_buf.at[bi, pl.ds(ch * EPV, EPV)],
                    out.at[pl.ds((core * 3 + ch) * VOL + vox, EPV)], osem)
                if start:
                    d.start()
                else:
                    d.wait()

        _ep_in(0, 0)

        def epi(k, _):
            bi = lax.rem(k, 2)
            pltpu.make_async_copy(acc.at[pl.ds(sbase + k * EPR, EPR)],
                                  acc_in.at[bi], esem).wait()

            @pl.when(k + 1 < NEP)
            def _():
                _ep_in(k + 1, lax.rem(k + 1, 2))

            @pl.when(k >= 2)
            def _():
                _ep_out(k - 2, bi, False)

            def norm(m, _):
                rr = m * 8 + i2
                ws = plsc.load_gather(acc_in.at[bi], [rr, p4 + 3])
                inv = 1.0 / (ws + 0.001)
                for ch in range(3):
                    s = plsc.load_gather(acc_in.at[bi], [rr, p4 + ch])
                    out_buf[bi, pl.ds(ch * EPV + m * 16, 16)] = s * inv
                return 0

            lax.fori_loop(0, EPV // 16, norm, 0)
            _ep_out(k, bi, True)
            return 0

        lax.fori_loop(0, NEP, epi, 0)
        _ep_out(NEP - 2, 0, False)
        _ep_out(NEP - 1, 1, False)
        plsc.subcore_barrier()
        return 0

    lax.fori_loop(0, NPASSES, one_pass, 0)


_sc_vox = functools.partial(
    pl.kernel,
    out_type=jax.ShapeDtypeStruct((B * 3 * VOL,), jnp.float32),
    mesh=plsc.VectorSubcoreMesh(core_axis_name="c", subcore_axis_name="s"),
    compiler_params=pltpu.CompilerParams(needs_layout_passes=False,
                                         use_tc_tiling_on_sc=False),
    scratch_types=[
        pltpu.VMEM((VPS,), jnp.float32),
        pltpu.VMEM((VPS,), jnp.float32),
        pltpu.VMEM((VPS,), jnp.float32),
        pltpu.VMEM((VPS,), jnp.float32),
        pltpu.VMEM((VPS,), jnp.float32),
        pltpu.VMEM((VPS,), jnp.float32),
        pltpu.VMEM((CPAD,), jnp.int32),
        pltpu.VMEM((VPS, 8), jnp.float32),
        pltpu.VMEM((RING, GROWS, 8), jnp.float32),
        pltpu.VMEM((NCHUNK, 128), jnp.int32),
        pltpu.VMEM((2, EPR, 8), jnp.float32),
        pltpu.VMEM((2, 3 * EPV), jnp.float32),
        pltpu.HBM((B * NS, CPAD, 8), jnp.float32),
        pltpu.SemaphoreType.DMA,
        pltpu.SemaphoreType.DMA,
        pltpu.SemaphoreType.DMA,
        pltpu.SemaphoreType.DMA,
        pltpu.SemaphoreType.DMA,
        pltpu.VMEM_SHARED((PROWS + 128, 8), jnp.float32),
    ],
)(_sc_body)


def kernel(smpl_vertices, smpl_vertex_code, smpl_face_code,
           smpl_face_indices, smpl_tetrahedron_indices):
    del smpl_face_code, smpl_face_indices, smpl_tetrahedron_indices
    pad = NVPAD - NV
    v = jnp.pad(smpl_vertices, ((0, 0), (0, pad), (0, 0)),
                constant_values=1e6)
    cde = jnp.pad(smpl_vertex_code, ((0, 0), (0, pad), (0, 0)))
    zeros_in = jnp.zeros((ZR, 8), jnp.float32)
    out = _sc_vox(v[:, :, 0].reshape(-1), v[:, :, 1].reshape(-1),
                  v[:, :, 2].reshape(-1), cde[:, :, 0].reshape(-1),
                  cde[:, :, 1].reshape(-1), cde[:, :, 2].reshape(-1),
                  zeros_in)
    return out.reshape(B, 3, RES, RES, RES)


# speedup vs baseline: 1.5866x; 1.5827x over previous
import functools

import jax
import jax.numpy as jnp
from jax import lax
from jax.experimental import pallas as pl
from jax.experimental.pallas import tpu as pltpu
from jax.experimental.pallas import tpu_sc as plsc

B = 2
NV = 6890
RES = 128
VOL = RES * RES * RES
SIG2 = 0.05 * 0.05
NS = 16
VPS = 432
NVPAD = NS * VPS
NVREG = VPS // 16
NCON = VPS * 27
NCHUNK = (NCON + 127) // 128
CPAD = NCHUNK * 128
GRP = 4
NGRP = NCHUNK // GRP
GROWS = GRP * 128
RING = 6
NPASSES = 8
PROWS = VOL // 2 // NPASSES
SROWS = PROWS // NS
EPR = 256
EPV = EPR * 2
NEP = SROWS // EPR
ZR = 2048

_OFFS = [(a, b, c) for a in (-1, 0, 1) for b in (-1, 0, 1) for c in (-1, 0, 1)]


def _floor_i32(x):
    t = x.astype(jnp.int32)
    return t - jnp.where(x < t.astype(jnp.float32), 1, 0).astype(jnp.int32)


def _sc_body(vx, vy, vz, c0, c1, c2, zeros_in, out,
             px, py, pz, q0, q1, q2, rows_all, stage, vbuf, lidx, acc_in,
             out_buf, vals_hbm, psem, ssem, zsem, esem, osem, acc):
    core = lax.axis_index("c")
    sub = lax.axis_index("s")
    wid = core * NS + sub
    vbase = core * NVPAD + sub * VPS
    iota = lax.iota(jnp.int32, 16)
    i2 = lax.shift_right_logical(iota, 1)
    p4 = (iota & 1) * 4

    pltpu.sync_copy(vx.at[pl.ds(vbase, VPS)], px)
    pltpu.sync_copy(vy.at[pl.ds(vbase, VPS)], py)
    pltpu.sync_copy(vz.at[pl.ds(vbase, VPS)], pz)
    pltpu.sync_copy(c0.at[pl.ds(vbase, VPS)], q0)
    pltpu.sync_copy(c1.at[pl.ds(vbase, VPS)], q1)
    pltpu.sync_copy(c2.at[pl.ds(vbase, VPS)], q2)

    for m in range((CPAD - NCON) // 16):
        rows_all[pl.ds(NCON + m * 16, 16)] = jnp.full((16,), 1 << 29,
                                                      jnp.int32)
    pltpu.sync_copy(zeros_in.at[pl.ds(0, CPAD - NCON)],
                    vals_hbm.at[wid, pl.ds(NCON, CPAD - NCON)])

    def gen(i, _):
        r16 = i * 16
        wx = px[pl.ds(r16, 16)]
        wy = py[pl.ds(r16, 16)]
        wz = pz[pl.ds(r16, 16)]
        a0 = q0[pl.ds(r16, 16)]
        a1 = q1[pl.ds(r16, 16)]
        a2 = q2[pl.ds(r16, 16)]
        bx = _floor_i32((wx * 0.5 + 0.5) * RES)
        by = _floor_i32((wy * 0.5 + 0.5) * RES)
        bz = _floor_i32((wz * 0.5 + 0.5) * RES)
        zero = jnp.zeros((16,), jnp.float32)
        for o, (oa, ob, oc) in enumerate(_OFFS):
            nx = jnp.clip(bx + oa, 0, RES - 1)
            ny = jnp.clip(by + ob, 0, RES - 1)
            nz = jnp.clip(bz + oc, 0, RES - 1)
            dx = (nx.astype(jnp.float32) + 0.5) * (2.0 / RES) - 1.0 - wx
            dy = (ny.astype(jnp.float32) + 0.5) * (2.0 / RES) - 1.0 - wy
            dz = (nz.astype(jnp.float32) + 0.5) * (2.0 / RES) - 1.0 - wz
            w = jnp.exp((dx * dx + dy * dy + dz * dz) * (-1.0 / SIG2))
            g = (nx * RES + ny) * RES + nz
            rows_all[pl.ds(i * VPS + o * 16, 16)] = (
                lax.shift_right_logical(g, 1))
            rvec = o * 16 + iota
            half = (g & 1) * 4
            anti = 4 - half
            for ch, val in enumerate((w * a0, w * a1, w * a2, w)):
                plsc.store_scatter(stage, [rvec, half + ch], val)
                plsc.store_scatter(stage, [rvec, anti + ch], zero)
        pltpu.sync_copy(stage, vals_hbm.at[wid, pl.ds(i * VPS, VPS)])
        return 0

    lax.fori_loop(0, NVREG, gen, 0)

    def one_pass(p, _):
        row_lo = p * PROWS
        for k in range(SROWS // ZR):
            pltpu.async_copy(zeros_in,
                             acc.at[pl.ds(sub * SROWS + k * ZR, ZR)], zsem)
        for k in range(SROWS // ZR):
            pltpu.make_async_copy(
                zeros_in, acc.at[pl.ds(sub * SROWS + k * ZR, ZR)],
                zsem).wait()

        def remap(j, _):
            for k in range(8):
                r = rows_all[pl.ds(j * 128 + k * 16, 16)]
                rel = r - row_lo
                match = (rel >= 0) & (rel < PROWS)
                trash = PROWS + k * 16 + iota
                lidx[j, pl.ds(k * 16, 16)] = jnp.where(match, rel, trash)
            return 0

        lax.fori_loop(0, NCHUNK, remap, 0)
        plsc.subcore_barrier()

        for r in range(RING - 1):
            pltpu.async_copy(vals_hbm.at[wid, pl.ds(r * GROWS, GROWS)],
                             vbuf.at[r], psem)

        def scat_grp(g, _):
            bi = lax.rem(g, RING)
            pltpu.make_async_copy(vals_hbm.at[wid, pl.ds(g * GROWS, GROWS)],
                                  vbuf.at[bi], psem).wait()
            for cc in range(GRP):
                pltpu.async_copy(vbuf.at[bi, pl.ds(cc * 128, 128)],
                                 acc.at[lidx.at[g * GRP + cc]], ssem,
                                 add=True)

            @pl.when(g >= 1)
            def _():
                pg = g - 1
                pbi = lax.rem(pg, RING)
                for cc in range(GRP):
                    pltpu.make_async_copy(
                        vbuf.at[pbi, pl.ds(cc * 128, 128)],
                        acc.at[lidx.at[pg * GRP + cc]], ssem).wait()

            @pl.when(g + RING - 1 < NGRP)
            def _():
                pltpu.async_copy(
                    vals_hbm.at[wid, pl.ds((g + RING - 1) * GROWS, GROWS)],
                    vbuf.at[lax.rem(g + RING - 1, RING)], psem)
            return 0

        lax.fori_loop(0, NGRP, scat_grp, 0)
        for cc in range(GRP):
            pltpu.make_async_copy(
                vbuf.at[(NGRP - 1) % RING, pl.ds(cc * 128, 128)],
                acc.at[lidx.at[(NGRP - 1) * GRP + cc]], ssem).wait()
        plsc.subcore_barrier()

        sbase = sub * SROWS

        def _ep_in(k, bi):
            pltpu.async_copy(acc.at[pl.ds(sbase + k * EPR, EPR)],
                             acc_in.at[bi], esem)

        def _ep_out(k, bi, start):
            vox = (row_lo + sbase + k * EPR) * 2
            for ch in range(3):
                d = pltpu.make_async_copy(
                    out_buf.at[bi, pl.ds(ch * EPV, EPV)],
                    out.at[pl.ds((core * 3 + ch) * VOL + vox, EPV)], osem)
                if start:
                    d.start()
                else:
                    d.wait()

        _ep_in(0, 0)

        def epi(k, _):
            bi = lax.rem(k, 2)
            pltpu.make_async_copy(acc.at[pl.ds(sbase + k * EPR, EPR)],
                                  acc_in.at[bi], esem).wait()

            @pl.when(k + 1 < NEP)
            def _():
                _ep_in(k + 1, lax.rem(k + 1, 2))

            @pl.when(k >= 2)
            def _():
                _ep_out(k - 2, bi, False)

            @plsc.parallel_loop(0, EPV // 16, unroll=4)
            def norm(m):
                rr = m * 8 + i2
                ws = plsc.load_gather(acc_in.at[bi], [rr, p4 + 3])
                inv = 1.0 / (ws + 0.001)
                for ch in range(3):
                    s = plsc.load_gather(acc_in.at[bi], [rr, p4 + ch])
                    out_buf[bi, pl.ds(ch * EPV + m * 16, 16)] = s * inv
            _ep_out(k, bi, True)
            return 0

        lax.fori_loop(0, NEP, epi, 0)
        _ep_out(NEP - 2, 0, False)
        _ep_out(NEP - 1, 1, False)
        plsc.subcore_barrier()
        return 0

    lax.fori_loop(0, NPASSES, one_pass, 0)


_sc_vox = functools.partial(
    pl.kernel,
    out_type=jax.ShapeDtypeStruct((B * 3 * VOL,), jnp.float32),
    mesh=plsc.VectorSubcoreMesh(core_axis_name="c", subcore_axis_name="s"),
    compiler_params=pltpu.CompilerParams(needs_layout_passes=False,
                                         use_tc_tiling_on_sc=False),
    scratch_types=[
        pltpu.VMEM((VPS,), jnp.float32),
        pltpu.VMEM((VPS,), jnp.float32),
        pltpu.VMEM((VPS,), jnp.float32),
        pltpu.VMEM((VPS,), jnp.float32),
        pltpu.VMEM((VPS,), jnp.float32),
        pltpu.VMEM((VPS,), jnp.float32),
        pltpu.VMEM((CPAD,), jnp.int32),
        pltpu.VMEM((VPS, 8), jnp.float32),
        pltpu.VMEM((RING, GROWS, 8), jnp.float32),
        pltpu.VMEM((NCHUNK, 128), jnp.int32),
        pltpu.VMEM((2, EPR, 8), jnp.float32),
        pltpu.VMEM((2, 3 * EPV), jnp.float32),
        pltpu.HBM((B * NS, CPAD, 8), jnp.float32),
        pltpu.SemaphoreType.DMA,
        pltpu.SemaphoreType.DMA,
        pltpu.SemaphoreType.DMA,
        pltpu.SemaphoreType.DMA,
        pltpu.SemaphoreType.DMA,
        pltpu.VMEM_SHARED((PROWS + 128, 8), jnp.float32),
    ],
)(_sc_body)


def kernel(smpl_vertices, smpl_vertex_code, smpl_face_code,
           smpl_face_indices, smpl_tetrahedron_indices):
    del smpl_face_code, smpl_face_indices, smpl_tetrahedron_indices
    pad = NVPAD - NV
    v = jnp.pad(smpl_vertices, ((0, 0), (0, pad), (0, 0)),
                constant_values=1e6)
    cde = jnp.pad(smpl_vertex_code, ((0, 0), (0, pad), (0, 0)))
    zeros_in = jnp.zeros((ZR, 8), jnp.float32)
    out = _sc_vox(v[:, :, 0].reshape(-1), v[:, :, 1].reshape(-1),
                  v[:, :, 2].reshape(-1), cde[:, :, 0].reshape(-1),
                  cde[:, :, 1].reshape(-1), cde[:, :, 2].reshape(-1),
                  zeros_in)
    return out.reshape(B, 3, RES, RES, RES)
